# baseline (speedup 1.0000x reference)
import functools

import jax
import jax.numpy as jnp
from jax import lax
from jax.experimental import pallas as pl
from jax.experimental.pallas import tpu as pltpu
from jax.experimental.pallas import tpu_sc as plsc

_NC = 2
_NS = 16
_N = 256
_D = 4 * _N


def _sc_build_table(months_hbm, ce_hbm, pe_hbm, mt_hbm, a_hbm,
                    ce_v, pe_v, midx_v, mrows_v, row_v, sem):
    wid = lax.axis_index("s") * _NC + lax.axis_index("c")
    r0 = wid * 2
    pltpu.sync_copy(ce_hbm, ce_v)
    pltpu.sync_copy(pe_hbm, pe_v)
    for k in range(2):
        r = r0 + k
        b = r // 32
        t = (r // 4) % 8
        s = r % 4
        pltpu.sync_copy(months_hbm.at[b], midx_v)
        pltpu.async_copy(mt_hbm.at[midx_v], mrows_v, sem).wait()
        zeros = jnp.zeros((16,), jnp.float32)
        for v in range(16):
            sl = pl.ds(v * 16, 16)
            row_v[pl.ds(0 * _N + v * 16, 16)] = ce_v[s, sl]
            row_v[pl.ds(1 * _N + v * 16, 16)] = pe_v[t, sl]
            row_v[pl.ds(2 * _N + v * 16, 16)] = mrows_v[t, sl]
            row_v[pl.ds(3 * _N + v * 16, 16)] = zeros
        pltpu.sync_copy(row_v, a_hbm.at[r])


def _build_additive_table(months, channel_embed, pos8, month_table):
    b, t = months.shape
    b_s = channel_embed.shape[0]
    builder = functools.partial(
        pl.kernel,
        out_type=jax.ShapeDtypeStruct((b * t * b_s, _D), jnp.float32),
        mesh=plsc.VectorSubcoreMesh(
            core_axis_name="c", subcore_axis_name="s",
            num_cores=_NC, num_subcores=_NS),
        scratch_types=[
            pltpu.VMEM((b_s, _N), jnp.float32),
            pltpu.VMEM((t, _N), jnp.float32),
            pltpu.VMEM((t,), jnp.int32),
            pltpu.VMEM((t, _N), jnp.float32),
            pltpu.VMEM((_D,), jnp.float32),
            pltpu.SemaphoreType.DMA,
        ],
    )(_sc_build_table)
    return builder(months, channel_embed, pos8, month_table)


def _tc_add_kernel(x_ref, a_ref, o_ref):
    o_ref[...] = x_ref[...] + a_ref[...][:, None]


def kernel(sensor_tokens, timestamps, channel_embed, pos_embed, month_table):
    b, h, w, t, b_s, d = sensor_tokens.shape
    hw = h * w
    br = 32
    x = sensor_tokens.reshape(b, hw, t, b_s, d)
    months = timestamps[:, :, 1].astype(jnp.int32)

    a = _build_additive_table(months, channel_embed, pos_embed[:t], month_table)
    a4 = a.reshape(b, t, b_s, d)

    out = pl.pallas_call(
        _tc_add_kernel,
        grid=(b, hw // br),
        in_specs=[
            pl.BlockSpec((1, br, t, b_s, d), lambda i, j: (i, j, 0, 0, 0)),
            pl.BlockSpec((1, t, b_s, d), lambda i, j: (i, 0, 0, 0)),
        ],
        out_specs=pl.BlockSpec((1, br, t, b_s, d), lambda i, j: (i, j, 0, 0, 0)),
        out_shape=jax.ShapeDtypeStruct(x.shape, x.dtype),
        compiler_params=pltpu.CompilerParams(
            dimension_semantics=("arbitrary", "arbitrary"),
        ),
    )(x, a4)
    return out.reshape(b, h, w, t, b_s, d)
